# baseline (speedup 1.0000x reference)
import functools

import jax
import jax.numpy as jnp
from jax import lax
from jax.experimental import pallas as pl
from jax.experimental.pallas import tpu as pltpu
from jax.experimental.pallas import tpu_sc as plsc

_THR = 0.1
_TOPK = 200
_B, _H, _W = 16, 512, 512
_IMG = _H * _W
_NW = 32
_HALF = _IMG // 2
_CAP = 4096
_PCAP = 512
_NBIS = 14
_NS = 8
_CHUNK = 16384
_LANES = 16


def _win7_lanes(x):
    neg = jnp.float32(-jnp.inf)

    def shl(a, s):
        return jnp.concatenate([a[:, s:], jnp.full((_H, s), neg, a.dtype)], axis=1)

    def shr(a, s):
        return jnp.concatenate([jnp.full((_H, s), neg, a.dtype), a[:, :-s]], axis=1)

    g1 = jnp.maximum(x, shl(x, 1))
    g2 = jnp.maximum(g1, shl(g1, 2))
    r1 = jnp.maximum(x, shr(x, 1))
    r2 = jnp.maximum(r1, shr(r1, 2))
    return jnp.maximum(g2, r2)


def _win7_sublanes(x):
    neg = jnp.float32(-jnp.inf)

    def shu(a, s):
        return jnp.concatenate([a[s:, :], jnp.full((s, _W), neg, a.dtype)], axis=0)

    def shd(a, s):
        return jnp.concatenate([jnp.full((s, _W), neg, a.dtype), a[:-s, :]], axis=0)

    g1 = jnp.maximum(x, shu(x, 1))
    g2 = jnp.maximum(g1, shu(g1, 2))
    r1 = jnp.maximum(x, shd(x, 1))
    r2 = jnp.maximum(r1, shd(r1, 2))
    return jnp.maximum(g2, r2)


def _nms_body(x_ref, out_ref):
    x = x_ref[0]
    p = _win7_sublanes(_win7_lanes(x))
    keep = (p > _THR) & (p == x)
    out_ref[0] = jnp.where(keep, x, 0.0)


_nms_call = pl.pallas_call(
    _nms_body,
    out_shape=jax.ShapeDtypeStruct((_B, _H, _W), jnp.float32),
    grid=(_B,),
    in_specs=[pl.BlockSpec((1, _H, _W), lambda i: (i, 0, 0))],
    out_specs=pl.BlockSpec((1, _H, _W), lambda i: (i, 0, 0)),
)


@functools.cache
def _make_compact():
    mesh = plsc.VectorSubcoreMesh(core_axis_name="c", subcore_axis_name="s")
    return pl.kernel(
        _compact_body,
        out_type=(
            jax.ShapeDtypeStruct((_B, 2 * _PCAP), jnp.float32),
            jax.ShapeDtypeStruct((_B, 2 * _PCAP), jnp.int32),
        ),
        mesh=mesh,
        scratch_types=[
            pltpu.VMEM((_CHUNK,), jnp.float32),
            pltpu.VMEM((_CHUNK,), jnp.float32),
            pltpu.VMEM((_CAP,), jnp.float32),
            pltpu.VMEM((_CAP,), jnp.int32),
            pltpu.VMEM((_PCAP,), jnp.float32),
            pltpu.VMEM((_PCAP,), jnp.int32),
            pltpu.SemaphoreType.DMA,
            pltpu.SemaphoreType.DMA,
        ],
        compiler_params=pltpu.CompilerParams(needs_layout_passes=False),
    )


def _compact_body(
    nms_hbm, ovals_hbm, oidx_hbm, buf0, buf1, cvals, cidx, pvals, pidx, sem0, sem1
):
    wid = lax.axis_index("s") * 2 + lax.axis_index("c")
    img = wid // 2
    half = wid % 2
    base = wid * _HALF
    local_base = half * _HALF

    zf = jnp.zeros((_LANES,), jnp.float32)
    zi = jnp.zeros((_LANES,), jnp.int32)
    ones = jnp.ones((_LANES,), jnp.int32)
    lane = lax.iota(jnp.int32, _LANES)

    def zero_body(j, carry):
        cvals[pl.ds(j * _LANES, _LANES)] = zf
        pvals[pl.ds(j * _LANES % _PCAP, _LANES)] = zf
        return carry

    lax.fori_loop(0, _CAP // _LANES, zero_body, 0, unroll=4)

    bufs = (buf0, buf1)
    sems = (sem0, sem1)
    n_chunks = _HALF // _CHUNK

    def start(c):
        return pltpu.async_copy(
            nms_hbm.at[pl.ds(base + c * _CHUNK, _CHUNK)], bufs[c % 2], sems[c % 2]
        )

    sv = _CHUNK // _LANES // _NS
    scap = _CAP // _NS
    step = jnp.full((_LANES,), _LANES, jnp.int32)
    stops = [jnp.full((_LANES,), s * scap + scap - 1 - _LANES, jnp.int32)
             for s in range(_NS)]
    pending = start(0)
    offs = [jnp.full((_LANES,), s * scap - 1, jnp.int32) for s in range(_NS)]
    for c in range(n_chunks):
        nxt = start(c + 1) if c + 1 < n_chunks else None
        pending.wait()
        cbuf = bufs[c % 2]
        idxs = [local_base + c * _CHUNK + s * sv * _LANES + lane for s in range(_NS)]

        def vec_body(j, carry, cbuf=cbuf):
            vs = [cbuf[pl.ds((s * sv + j) * _LANES, _LANES)] for s in range(_NS)]
            msks = [v > 0.0 for v in vs]
            sels = [jnp.where(m, ones, zi) for m in msks]
            pcs = [plsc.cumsum(sel) for sel in sels]
            poss = [carry[s][0] + pcs[s] for s in range(_NS)]
            for s in range(_NS):
                plsc.store_scatter(cvals, [poss[s]], vs[s], mask=msks[s])
            for s in range(_NS):
                plsc.store_scatter(cidx, [poss[s]], carry[s][1], mask=msks[s])
            cnts = [plsc.all_reduce_population_count(m) for m in msks]
            return tuple(
                (jnp.minimum(carry[s][0] + cnts[s], stops[s]), carry[s][1] + step)
                for s in range(_NS)
            )

        carry = tuple((offs[s], idxs[s]) for s in range(_NS))
        carry = lax.fori_loop(0, sv, vec_body, carry, unroll=2)
        offs = [carry[s][0] for s in range(_NS)]
        pending = nxt

    lo = jnp.zeros((_LANES,), jnp.float32)
    hi = jnp.ones((_LANES,), jnp.float32)
    target = jnp.full((_LANES,), 256, jnp.int32)

    nv = _CAP // _LANES // _NS

    def bis_body(i, carry):
        lo, hi = carry
        mid = (lo + hi) * 0.5

        def cnt_body(j, accs):
            vs = [cvals[pl.ds((s * nv + j) * _LANES, _LANES)] for s in range(_NS)]
            ms = [v >= mid for v in vs]
            ps = [plsc.all_reduce_population_count(m) for m in ms]
            return tuple(accs[s] + ps[s] for s in range(_NS))

        accs = lax.fori_loop(0, nv, cnt_body, (zi,) * _NS, unroll=2)
        cnt = sum(accs[1:], accs[0])
        ge = cnt >= target
        return jnp.where(ge, mid, lo), jnp.where(ge, hi, mid)

    lo, hi = lax.fori_loop(0, _NBIS, bis_body, (lo, hi))

    pcap_m1 = jnp.full((_LANES,), _PCAP - 1, jnp.int32)

    def sel_body(j, off_m1):
        v = cvals[pl.ds(j * _LANES, _LANES)]
        ix = cidx[pl.ds(j * _LANES, _LANES)]
        msk = (v >= lo) & (v > 0.0)
        pc = plsc.cumsum(jnp.where(msk, ones, zi))
        pos = jnp.minimum(off_m1 + pc, pcap_m1)
        plsc.store_scatter(pvals, [pos], v, mask=msk)
        plsc.store_scatter(pidx, [pos], ix, mask=msk)
        return off_m1 + plsc.all_reduce_population_count(msk)

    lax.fori_loop(0, _CAP // _LANES, sel_body, -ones, unroll=8)

    pltpu.sync_copy(pvals, ovals_hbm.at[img, pl.ds(half * _PCAP, _PCAP)])
    pltpu.sync_copy(pidx, oidx_hbm.at[img, pl.ds(half * _PCAP, _PCAP)])


def _select_body(vals_ref, idx_ref, coords_ref, probs_ref, v_scr, i_acc):
    v_scr[...] = vals_ref[...]
    big = jnp.int32(1 << 30)
    col = lax.broadcasted_iota(jnp.int32, (_B, _TOPK), 1)
    neg1 = jnp.float32(-1.0)
    one = jnp.int32(1)
    zero = jnp.int32(0)
    m0 = jnp.max(v_scr[...], axis=1, keepdims=True)
    ones_col = jnp.ones((2 * _PCAP, 1), jnp.float32)

    def body(r, m):
        v = v_scr[...]
        tie = v == m
        idxs = idx_ref[...]
        imin = jnp.min(jnp.where(tie, idxs, big), axis=1, keepdims=True)
        m2 = jnp.max(jnp.where(tie, neg1, v), axis=1, keepdims=True)
        tie_f = jnp.where(tie, jnp.float32(1.0), jnp.float32(0.0))
        ntie = jax.lax.dot_general(
            tie_f, ones_col, (((1,), (0,)), ((), ())),
            preferred_element_type=jnp.float32,
        )
        hit = col == r
        probs_ref[...] = jnp.where(hit, m, probs_ref[...])
        i_acc[...] = jnp.where(hit, imin, i_acc[...])
        v_scr[...] = jnp.where(tie & (idxs == imin), neg1, v)
        return jnp.where(ntie > jnp.float32(1.0), m, m2)

    lax.fori_loop(0, _TOPK, body, m0)
    packed = i_acc[...]
    coords_ref[0] = packed // _W
    coords_ref[1] = packed % _W


_select_call = pl.pallas_call(
    _select_body,
    out_shape=(
        jax.ShapeDtypeStruct((2, _B, _TOPK), jnp.int32),
        jax.ShapeDtypeStruct((_B, _TOPK), jnp.float32),
    ),
    scratch_shapes=[
        pltpu.VMEM((_B, 2 * _PCAP), jnp.float32),
        pltpu.VMEM((_B, _TOPK), jnp.int32),
    ],
)


@jax.jit
def kernel(center_map):
    x = center_map.reshape(_B, _H, _W)
    nms = _nms_call(x)
    vals, idx = _make_compact()(nms.reshape(_B * _IMG))
    coords2, probs = _select_call(vals, idx)
    coords = jnp.stack([coords2[0], coords2[1]], axis=-1)
    return coords, probs


# speedup vs baseline: 1.0867x; 1.0867x over previous
import functools

import jax
import jax.numpy as jnp
from jax import lax
from jax.experimental import pallas as pl
from jax.experimental.pallas import tpu as pltpu
from jax.experimental.pallas import tpu_sc as plsc

_THR = 0.1
_TOPK = 200
_B, _H, _W = 16, 512, 512
_IMG = _H * _W
_NW = 32
_HALF = _IMG // 2
_CAP = 4096
_PCAP = 512
_NBIS = 14
_NS = 8
_CHUNK = 16384
_LANES = 16


def _win7_lanes(x):
    neg = jnp.float32(-jnp.inf)

    def shl(a, s):
        return jnp.concatenate([a[:, s:], jnp.full((_H, s), neg, a.dtype)], axis=1)

    def shr(a, s):
        return jnp.concatenate([jnp.full((_H, s), neg, a.dtype), a[:, :-s]], axis=1)

    g1 = jnp.maximum(x, shl(x, 1))
    g2 = jnp.maximum(g1, shl(g1, 2))
    r1 = jnp.maximum(x, shr(x, 1))
    r2 = jnp.maximum(r1, shr(r1, 2))
    return jnp.maximum(g2, r2)


def _win7_sublanes(x):
    neg = jnp.float32(-jnp.inf)

    def shu(a, s):
        return jnp.concatenate([a[s:, :], jnp.full((s, _W), neg, a.dtype)], axis=0)

    def shd(a, s):
        return jnp.concatenate([jnp.full((s, _W), neg, a.dtype), a[:-s, :]], axis=0)

    g1 = jnp.maximum(x, shu(x, 1))
    g2 = jnp.maximum(g1, shu(g1, 2))
    r1 = jnp.maximum(x, shd(x, 1))
    r2 = jnp.maximum(r1, shd(r1, 2))
    return jnp.maximum(g2, r2)


def _nms_body(x_ref, out_ref):
    x = x_ref[0]
    p = _win7_sublanes(_win7_lanes(x))
    keep = (p > _THR) & (p == x)
    out_ref[0] = jnp.where(keep, x, 0.0)


_nms_call = pl.pallas_call(
    _nms_body,
    out_shape=jax.ShapeDtypeStruct((_B, _H, _W), jnp.float32),
    grid=(_B,),
    in_specs=[pl.BlockSpec((1, _H, _W), lambda i: (i, 0, 0))],
    out_specs=pl.BlockSpec((1, _H, _W), lambda i: (i, 0, 0)),
)


@functools.cache
def _make_compact():
    mesh = plsc.VectorSubcoreMesh(core_axis_name="c", subcore_axis_name="s")
    return pl.kernel(
        _compact_body,
        out_type=(
            jax.ShapeDtypeStruct((_B, 2 * _PCAP), jnp.float32),
            jax.ShapeDtypeStruct((_B, 2 * _PCAP), jnp.int32),
        ),
        mesh=mesh,
        scratch_types=[
            pltpu.VMEM((_CHUNK,), jnp.float32),
            pltpu.VMEM((_CHUNK,), jnp.float32),
            pltpu.VMEM((_CAP,), jnp.float32),
            pltpu.VMEM((_CAP,), jnp.int32),
            pltpu.VMEM((_PCAP,), jnp.float32),
            pltpu.VMEM((_PCAP,), jnp.int32),
            pltpu.SemaphoreType.DMA,
            pltpu.SemaphoreType.DMA,
        ],
        compiler_params=pltpu.CompilerParams(needs_layout_passes=False),
    )


def _compact_body(
    nms_hbm, ovals_hbm, oidx_hbm, buf0, buf1, cvals, cidx, pvals, pidx, sem0, sem1
):
    wid = lax.axis_index("s") * 2 + lax.axis_index("c")
    img = wid // 2
    half = wid % 2
    base = wid * _HALF
    local_base = half * _HALF

    zf = jnp.zeros((_LANES,), jnp.float32)
    zi = jnp.zeros((_LANES,), jnp.int32)
    ones = jnp.ones((_LANES,), jnp.int32)
    lane = lax.iota(jnp.int32, _LANES)

    def zero_body(j, carry):
        cvals[pl.ds(j * _LANES, _LANES)] = zf
        pvals[pl.ds(j * _LANES % _PCAP, _LANES)] = zf
        return carry

    lax.fori_loop(0, _CAP // _LANES, zero_body, 0, unroll=4)

    bufs = (buf0, buf1)
    sems = (sem0, sem1)
    n_chunks = _HALF // _CHUNK

    def start(c):
        return pltpu.async_copy(
            nms_hbm.at[pl.ds(base + c * _CHUNK, _CHUNK)], bufs[c % 2], sems[c % 2]
        )

    sv = _CHUNK // _LANES // _NS
    scap = _CAP // _NS
    step = jnp.full((_LANES,), _LANES, jnp.int32)
    stops = [jnp.full((_LANES,), s * scap + scap - 1 - _LANES, jnp.int32)
             for s in range(_NS)]
    pending = start(0)
    offs = [jnp.full((_LANES,), s * scap - 1, jnp.int32) for s in range(_NS)]
    for c in range(n_chunks):
        nxt = start(c + 1) if c + 1 < n_chunks else None
        pending.wait()
        cbuf = bufs[c % 2]
        idxs = [local_base + c * _CHUNK + s * sv * _LANES + lane for s in range(_NS)]

        def vec_body(j, carry, cbuf=cbuf):
            vs = [cbuf[pl.ds((s * sv + j) * _LANES, _LANES)] for s in range(_NS)]
            msks = [v > 0.0 for v in vs]
            sels = [jnp.where(m, ones, zi) for m in msks]
            pcs = [plsc.cumsum(sel) for sel in sels]
            poss = [carry[s][0] + pcs[s] for s in range(_NS)]
            for s in range(_NS):
                plsc.store_scatter(cvals, [poss[s]], vs[s], mask=msks[s])
            for s in range(_NS):
                plsc.store_scatter(cidx, [poss[s]], carry[s][1], mask=msks[s])
            cnts = [plsc.all_reduce_population_count(m) for m in msks]
            return tuple(
                (jnp.minimum(carry[s][0] + cnts[s], stops[s]), carry[s][1] + step)
                for s in range(_NS)
            )

        carry = tuple((offs[s], idxs[s]) for s in range(_NS))
        carry = lax.fori_loop(0, sv, vec_body, carry, unroll=4)
        offs = [carry[s][0] for s in range(_NS)]
        pending = nxt

    lo = jnp.zeros((_LANES,), jnp.float32)
    hi = jnp.ones((_LANES,), jnp.float32)
    target = jnp.full((_LANES,), 256, jnp.int32)

    nv = _CAP // _LANES // _NS

    def bis_body(i, carry):
        lo, hi = carry
        mid = (lo + hi) * 0.5

        def cnt_body(j, accs):
            vs = [cvals[pl.ds((s * nv + j) * _LANES, _LANES)] for s in range(_NS)]
            ms = [v >= mid for v in vs]
            ps = [plsc.all_reduce_population_count(m) for m in ms]
            return tuple(accs[s] + ps[s] for s in range(_NS))

        accs = lax.fori_loop(0, nv, cnt_body, (zi,) * _NS, unroll=2)
        cnt = sum(accs[1:], accs[0])
        ge = cnt >= target
        return jnp.where(ge, mid, lo), jnp.where(ge, hi, mid)

    lo, hi = lax.fori_loop(0, _NBIS, bis_body, (lo, hi))

    pcap_m1 = jnp.full((_LANES,), _PCAP - 1, jnp.int32)

    def sel_body(j, off_m1):
        v = cvals[pl.ds(j * _LANES, _LANES)]
        ix = cidx[pl.ds(j * _LANES, _LANES)]
        msk = (v >= lo) & (v > 0.0)
        pc = plsc.cumsum(jnp.where(msk, ones, zi))
        pos = jnp.minimum(off_m1 + pc, pcap_m1)
        plsc.store_scatter(pvals, [pos], v, mask=msk)
        plsc.store_scatter(pidx, [pos], ix, mask=msk)
        return off_m1 + plsc.all_reduce_population_count(msk)

    lax.fori_loop(0, _CAP // _LANES, sel_body, -ones, unroll=8)

    pltpu.sync_copy(pvals, ovals_hbm.at[img, pl.ds(half * _PCAP, _PCAP)])
    pltpu.sync_copy(pidx, oidx_hbm.at[img, pl.ds(half * _PCAP, _PCAP)])


def _select_body(vals_ref, idx_ref, coords_ref, probs_ref, v_scr, i_acc):
    v_scr[...] = vals_ref[...]
    big = jnp.int32(1 << 30)
    col = lax.broadcasted_iota(jnp.int32, (_B, _TOPK), 1)
    neg1 = jnp.float32(-1.0)
    one = jnp.int32(1)
    zero = jnp.int32(0)
    m0 = jnp.max(v_scr[...], axis=1, keepdims=True)

    def body(r, m):
        v = v_scr[...]
        tie = v == m
        idxs = idx_ref[...]
        imin = jnp.min(jnp.where(tie, idxs, big), axis=1, keepdims=True)
        ntie = jnp.sum(jnp.where(tie, one, zero), axis=1, keepdims=True)
        m2 = jnp.max(jnp.where(tie, neg1, v), axis=1, keepdims=True)
        hit = col == r
        probs_ref[...] = jnp.where(hit, m, probs_ref[...])
        i_acc[...] = jnp.where(hit, imin, i_acc[...])
        v_scr[...] = jnp.where(tie & (idxs == imin), neg1, v)
        return jnp.where(ntie > 1, m, m2)

    lax.fori_loop(0, _TOPK, body, m0)
    packed = i_acc[...]
    coords_ref[0] = packed // _W
    coords_ref[1] = packed % _W


_select_call = pl.pallas_call(
    _select_body,
    out_shape=(
        jax.ShapeDtypeStruct((2, _B, _TOPK), jnp.int32),
        jax.ShapeDtypeStruct((_B, _TOPK), jnp.float32),
    ),
    scratch_shapes=[
        pltpu.VMEM((_B, 2 * _PCAP), jnp.float32),
        pltpu.VMEM((_B, _TOPK), jnp.int32),
    ],
)


@jax.jit
def kernel(center_map):
    x = center_map.reshape(_B, _H, _W)
    nms = _nms_call(x)
    vals, idx = _make_compact()(nms.reshape(_B * _IMG))
    coords2, probs = _select_call(vals, idx)
    coords = jnp.stack([coords2[0], coords2[1]], axis=-1)
    return coords, probs


# speedup vs baseline: 1.1012x; 1.0133x over previous
import functools

import jax
import jax.numpy as jnp
from jax import lax
from jax.experimental import pallas as pl
from jax.experimental.pallas import tpu as pltpu
from jax.experimental.pallas import tpu_sc as plsc

_THR = 0.1
_TOPK = 200
_B, _H, _W = 16, 512, 512
_IMG = _H * _W
_NW = 32
_HALF = _IMG // 2
_CAP = 4096
_PCAP = 512
_NBIS = 14
_NS = 8
_CHUNK = 16384
_LANES = 16


def _win7_lanes(x):
    neg = jnp.float32(-jnp.inf)

    def shl(a, s):
        return jnp.concatenate([a[:, s:], jnp.full((_H, s), neg, a.dtype)], axis=1)

    def shr(a, s):
        return jnp.concatenate([jnp.full((_H, s), neg, a.dtype), a[:, :-s]], axis=1)

    g1 = jnp.maximum(x, shl(x, 1))
    g2 = jnp.maximum(g1, shl(g1, 2))
    r1 = jnp.maximum(x, shr(x, 1))
    r2 = jnp.maximum(r1, shr(r1, 2))
    return jnp.maximum(g2, r2)


def _win7_sublanes(x):
    neg = jnp.float32(-jnp.inf)

    def shu(a, s):
        return jnp.concatenate([a[s:, :], jnp.full((s, _W), neg, a.dtype)], axis=0)

    def shd(a, s):
        return jnp.concatenate([jnp.full((s, _W), neg, a.dtype), a[:-s, :]], axis=0)

    g1 = jnp.maximum(x, shu(x, 1))
    g2 = jnp.maximum(g1, shu(g1, 2))
    r1 = jnp.maximum(x, shd(x, 1))
    r2 = jnp.maximum(r1, shd(r1, 2))
    return jnp.maximum(g2, r2)


def _nms_body(x_ref, out_ref):
    x = x_ref[0]
    p = _win7_sublanes(_win7_lanes(x))
    keep = (p > _THR) & (p == x)
    out_ref[0] = jnp.where(keep, x, 0.0)


_nms_call = pl.pallas_call(
    _nms_body,
    out_shape=jax.ShapeDtypeStruct((_B, _H, _W), jnp.float32),
    grid=(_B,),
    in_specs=[pl.BlockSpec((1, _H, _W), lambda i: (i, 0, 0))],
    out_specs=pl.BlockSpec((1, _H, _W), lambda i: (i, 0, 0)),
)


@functools.cache
def _make_compact():
    mesh = plsc.VectorSubcoreMesh(core_axis_name="c", subcore_axis_name="s")
    return pl.kernel(
        _compact_body,
        out_type=(
            jax.ShapeDtypeStruct((_B, 2 * _PCAP), jnp.float32),
            jax.ShapeDtypeStruct((_B, 2 * _PCAP), jnp.int32),
        ),
        mesh=mesh,
        scratch_types=[
            pltpu.VMEM((_CHUNK,), jnp.float32),
            pltpu.VMEM((_CHUNK,), jnp.float32),
            pltpu.VMEM((_CAP,), jnp.float32),
            pltpu.VMEM((_CAP,), jnp.int32),
            pltpu.VMEM((_PCAP,), jnp.float32),
            pltpu.VMEM((_PCAP,), jnp.int32),
            pltpu.SemaphoreType.DMA,
            pltpu.SemaphoreType.DMA,
        ],
        compiler_params=pltpu.CompilerParams(needs_layout_passes=False),
    )


def _compact_body(
    nms_hbm, ovals_hbm, oidx_hbm, buf0, buf1, cvals, cidx, pvals, pidx, sem0, sem1
):
    wid = lax.axis_index("s") * 2 + lax.axis_index("c")
    img = wid // 2
    half = wid % 2
    base = wid * _HALF
    local_base = half * _HALF

    zf = jnp.zeros((_LANES,), jnp.float32)
    zi = jnp.zeros((_LANES,), jnp.int32)
    ones = jnp.ones((_LANES,), jnp.int32)
    lane = lax.iota(jnp.int32, _LANES)

    def zero_body(j, carry):
        cvals[pl.ds(j * _LANES, _LANES)] = zf
        pvals[pl.ds(j * _LANES % _PCAP, _LANES)] = zf
        return carry

    lax.fori_loop(0, _CAP // _LANES, zero_body, 0, unroll=4)

    bufs = (buf0, buf1)
    sems = (sem0, sem1)
    n_chunks = _HALF // _CHUNK

    def start(c):
        return pltpu.async_copy(
            nms_hbm.at[pl.ds(base + c * _CHUNK, _CHUNK)], bufs[c % 2], sems[c % 2]
        )

    sv = _CHUNK // _LANES // _NS
    scap = _CAP // _NS
    step = jnp.full((_LANES,), _LANES, jnp.int32)
    stops = [jnp.full((_LANES,), s * scap + scap - 1 - _LANES, jnp.int32)
             for s in range(_NS)]
    pending = start(0)
    offs = [jnp.full((_LANES,), s * scap - 1, jnp.int32) for s in range(_NS)]
    for c in range(n_chunks):
        nxt = start(c + 1) if c + 1 < n_chunks else None
        pending.wait()
        cbuf = bufs[c % 2]
        idxs = [local_base + c * _CHUNK + s * sv * _LANES + lane for s in range(_NS)]

        def vec_body(j, carry, cbuf=cbuf):
            vs = [cbuf[pl.ds((s * sv + j) * _LANES, _LANES)] for s in range(_NS)]
            msks = [v > 0.0 for v in vs]
            sels = [jnp.where(m, ones, zi) for m in msks]
            pcs = [plsc.cumsum(sel) for sel in sels]
            poss = [carry[s][0] + pcs[s] for s in range(_NS)]
            for s in range(_NS):
                plsc.store_scatter(cvals, [poss[s]], vs[s], mask=msks[s])
            for s in range(_NS):
                plsc.store_scatter(cidx, [poss[s]], carry[s][1], mask=msks[s])
            cnts = [plsc.all_reduce_population_count(m) for m in msks]
            return tuple(
                (jnp.minimum(carry[s][0] + cnts[s], stops[s]), carry[s][1] + step)
                for s in range(_NS)
            )

        carry = tuple((offs[s], idxs[s]) for s in range(_NS))
        carry = lax.fori_loop(0, sv, vec_body, carry, unroll=2)
        offs = [carry[s][0] for s in range(_NS)]
        pending = nxt

    lo = jnp.zeros((_LANES,), jnp.float32)
    hi = jnp.ones((_LANES,), jnp.float32)
    target = jnp.full((_LANES,), 256, jnp.int32)

    nv = _CAP // _LANES // _NS

    def bis_body(i, carry):
        lo, hi = carry
        mid = (lo + hi) * 0.5

        def cnt_body(j, accs):
            vs = [cvals[pl.ds((s * nv + j) * _LANES, _LANES)] for s in range(_NS)]
            ms = [v >= mid for v in vs]
            ps = [plsc.all_reduce_population_count(m) for m in ms]
            return tuple(accs[s] + ps[s] for s in range(_NS))

        accs = lax.fori_loop(0, nv, cnt_body, (zi,) * _NS, unroll=2)
        cnt = sum(accs[1:], accs[0])
        ge = cnt >= target
        return jnp.where(ge, mid, lo), jnp.where(ge, hi, mid)

    lo, hi = lax.fori_loop(0, _NBIS, bis_body, (lo, hi))

    pcap_m1 = jnp.full((_LANES,), _PCAP - 1, jnp.int32)

    def sel_body(j, off_m1):
        v = cvals[pl.ds(j * _LANES, _LANES)]
        ix = cidx[pl.ds(j * _LANES, _LANES)]
        msk = (v >= lo) & (v > 0.0)
        pc = plsc.cumsum(jnp.where(msk, ones, zi))
        pos = jnp.minimum(off_m1 + pc, pcap_m1)
        plsc.store_scatter(pvals, [pos], v, mask=msk)
        plsc.store_scatter(pidx, [pos], ix, mask=msk)
        return off_m1 + plsc.all_reduce_population_count(msk)

    lax.fori_loop(0, _CAP // _LANES, sel_body, -ones, unroll=8)

    pltpu.sync_copy(pvals, ovals_hbm.at[img, pl.ds(half * _PCAP, _PCAP)])
    pltpu.sync_copy(pidx, oidx_hbm.at[img, pl.ds(half * _PCAP, _PCAP)])


def _select_body(vals_ref, idx_ref, coords_ref, probs_ref, v_scr, i_acc):
    v_scr[...] = vals_ref[...]
    big = jnp.int32(1 << 30)
    col = lax.broadcasted_iota(jnp.int32, (_B, _TOPK), 1)
    neg1 = jnp.float32(-1.0)
    one = jnp.int32(1)
    zero = jnp.int32(0)
    m0 = jnp.max(v_scr[...], axis=1, keepdims=True)

    def body(r, m):
        v = v_scr[...]
        tie = v == m
        idxs = idx_ref[...]
        imin = jnp.min(jnp.where(tie, idxs, big), axis=1, keepdims=True)
        ntie = jnp.sum(jnp.where(tie, one, zero), axis=1, keepdims=True)
        m2 = jnp.max(jnp.where(tie, neg1, v), axis=1, keepdims=True)
        hit = col == r
        probs_ref[...] = jnp.where(hit, m, probs_ref[...])
        i_acc[...] = jnp.where(hit, imin, i_acc[...])
        v_scr[...] = jnp.where(tie & (idxs == imin), neg1, v)
        return jnp.where(ntie > 1, m, m2)

    lax.fori_loop(0, _TOPK, body, m0)
    packed = i_acc[...]
    coords_ref[0] = packed // _W
    coords_ref[1] = packed % _W


_select_call = pl.pallas_call(
    _select_body,
    out_shape=(
        jax.ShapeDtypeStruct((2, _B, _TOPK), jnp.int32),
        jax.ShapeDtypeStruct((_B, _TOPK), jnp.float32),
    ),
    scratch_shapes=[
        pltpu.VMEM((_B, 2 * _PCAP), jnp.float32),
        pltpu.VMEM((_B, _TOPK), jnp.int32),
    ],
)


@jax.jit
def kernel(center_map):
    x = center_map.reshape(_B, _H, _W)
    nms = _nms_call(x)
    vals, idx = _make_compact()(nms.reshape(_B * _IMG))
    coords2, probs = _select_call(vals, idx)
    coords = jnp.stack([coords2[0], coords2[1]], axis=-1)
    return coords, probs
